# baseline (speedup 1.0000x reference)
import functools

import jax
import jax.numpy as jnp
from jax.experimental import pallas as pl
from jax.experimental.pallas import tpu as pltpu

NUM_EXPERT = 8
TOP_K = 2
D_MODEL = 768
D_HIDDEN = 768
N_TOKENS = 4096

TOKEN_TILE = 1024


def _moe_body(x_ref, wg_ref, bg_ref, w1a_ref, w1b_ref, w1c_ref, w1d_ref,
              b1_ref, w2a_ref, w2b_ref, w2c_ref, w2d_ref, b2_ref, out_ref):
    w1_refs = (w1a_ref, w1b_ref, w1c_ref, w1d_ref)
    w2_refs = (w2a_ref, w2b_ref, w2c_ref, w2d_ref)
    x = x_ref[...]

    logits = (
        jnp.dot(x, wg_ref[...], preferred_element_type=jnp.float32)
        + bg_ref[...]
    )
    e_iota = jax.lax.broadcasted_iota(jnp.int32, logits.shape, 1)
    i1 = jnp.argmax(logits, axis=-1)[:, None]
    v1 = jnp.max(logits, axis=-1, keepdims=True)
    masked = jnp.where(e_iota == i1, -jnp.inf, logits)
    i2 = jnp.argmax(masked, axis=-1)[:, None]
    v2 = jnp.max(masked, axis=-1, keepdims=True)
    d = jnp.exp(v2 - v1)
    s1 = 1.0 / (1.0 + d)
    s2 = d / (1.0 + d)
    w = jnp.where(e_iota == i1, s1, 0.0) + jnp.where(e_iota == i2, s2, 0.0)

    acc = jnp.zeros(x.shape, dtype=jnp.float32)
    xb = x.astype(jnp.bfloat16)
    for e in range(NUM_EXPERT):
        h = jnp.dot(xb, w1_refs[e // 2][e % 2].astype(jnp.bfloat16),
                    preferred_element_type=jnp.float32)
        h = jnp.maximum(h + b1_ref[e], 0.0)
        y = jnp.dot(h.astype(jnp.bfloat16),
                    w2_refs[e // 2][e % 2].astype(jnp.bfloat16),
                    preferred_element_type=jnp.float32)
        y = y + b2_ref[e]
        acc = acc + w[:, e][:, None] * y
    out_ref[...] = acc


@jax.jit
def kernel(moe_inp, Wg, bg, W1, b1, W2, b2):
    n = moe_inp.shape[0]
    grid = (n // TOKEN_TILE,)
    bg2 = bg.reshape(1, NUM_EXPERT)
    return pl.pallas_call(
        _moe_body,
        grid=grid,
        in_specs=[
            pl.BlockSpec((TOKEN_TILE, D_MODEL), lambda i: (i, 0)),
            pl.BlockSpec((D_MODEL, NUM_EXPERT), lambda i: (0, 0)),
            pl.BlockSpec((1, NUM_EXPERT), lambda i: (0, 0)),
            pl.BlockSpec((2, D_MODEL, D_HIDDEN), lambda i: (0, 0, 0)),
            pl.BlockSpec((2, D_MODEL, D_HIDDEN), lambda i: (1, 0, 0)),
            pl.BlockSpec((2, D_MODEL, D_HIDDEN), lambda i: (2, 0, 0)),
            pl.BlockSpec((2, D_MODEL, D_HIDDEN), lambda i: (3, 0, 0)),
            pl.BlockSpec((NUM_EXPERT, D_HIDDEN), lambda i: (0, 0)),
            pl.BlockSpec((2, D_HIDDEN, D_MODEL), lambda i: (0, 0, 0)),
            pl.BlockSpec((2, D_HIDDEN, D_MODEL), lambda i: (1, 0, 0)),
            pl.BlockSpec((2, D_HIDDEN, D_MODEL), lambda i: (2, 0, 0)),
            pl.BlockSpec((2, D_HIDDEN, D_MODEL), lambda i: (3, 0, 0)),
            pl.BlockSpec((NUM_EXPERT, D_MODEL), lambda i: (0, 0)),
        ],
        out_specs=pl.BlockSpec((TOKEN_TILE, D_MODEL), lambda i: (i, 0)),
        out_shape=jax.ShapeDtypeStruct((n, D_MODEL), jnp.float32),
    )(moe_inp, Wg, bg2, W1, W1, W1, W1, b1, W2, W2, W2, W2, b2)


# speedup vs baseline: 1.0022x; 1.0022x over previous
import jax
import jax.numpy as jnp
from jax.experimental import pallas as pl

NUM_EXPERT = 8
TOP_K = 2
D_MODEL = 768
D_HIDDEN = 768
N_TOKENS = 4096

TOKEN_TILE = 1024


def _moe_body(x_ref, wg_ref, bg_ref, w1_ref, b1_ref, w2_ref, b2_ref, out_ref):
    x = x_ref[...]

    logits = (
        jnp.dot(x, wg_ref[...], preferred_element_type=jnp.float32)
        + bg_ref[...]
    )
    e_iota = jax.lax.broadcasted_iota(jnp.int32, logits.shape, 1)
    i1 = jnp.argmax(logits, axis=-1)[:, None]
    v1 = jnp.max(logits, axis=-1, keepdims=True)
    masked = jnp.where(e_iota == i1, -jnp.inf, logits)
    i2 = jnp.argmax(masked, axis=-1)[:, None]
    v2 = jnp.max(masked, axis=-1, keepdims=True)
    d = jnp.exp(v2 - v1)
    s1 = 1.0 / (1.0 + d)
    s2 = d / (1.0 + d)
    w = jnp.where(e_iota == i1, s1, 0.0) + jnp.where(e_iota == i2, s2, 0.0)

    acc = jnp.zeros(x.shape, dtype=jnp.float32)
    xb = x.astype(jnp.bfloat16)
    for e in range(NUM_EXPERT):
        h = jnp.dot(xb, w1_ref[e].astype(jnp.bfloat16),
                    preferred_element_type=jnp.float32)
        h = jnp.maximum(h + b1_ref[e], 0.0)
        y = jnp.dot(h.astype(jnp.bfloat16), w2_ref[e].astype(jnp.bfloat16),
                    preferred_element_type=jnp.float32)
        y = y + b2_ref[e]
        acc = acc + w[:, e][:, None] * y
    out_ref[...] = acc


@jax.jit
def kernel(moe_inp, Wg, bg, W1, b1, W2, b2):
    n = moe_inp.shape[0]
    grid = (n // TOKEN_TILE,)
    bg2 = bg.reshape(1, NUM_EXPERT)
    return pl.pallas_call(
        _moe_body,
        grid=grid,
        in_specs=[
            pl.BlockSpec((TOKEN_TILE, D_MODEL), lambda i: (i, 0)),
            pl.BlockSpec((D_MODEL, NUM_EXPERT), lambda i: (0, 0)),
            pl.BlockSpec((1, NUM_EXPERT), lambda i: (0, 0)),
            pl.BlockSpec((NUM_EXPERT, D_MODEL, D_HIDDEN), lambda i: (0, 0, 0)),
            pl.BlockSpec((NUM_EXPERT, D_HIDDEN), lambda i: (0, 0)),
            pl.BlockSpec((NUM_EXPERT, D_HIDDEN, D_MODEL), lambda i: (0, 0, 0)),
            pl.BlockSpec((NUM_EXPERT, D_MODEL), lambda i: (0, 0)),
        ],
        out_specs=pl.BlockSpec((TOKEN_TILE, D_MODEL), lambda i: (i, 0)),
        out_shape=jax.ShapeDtypeStruct((n, D_MODEL), jnp.float32),
    )(moe_inp, Wg, bg2, W1, b1, W2, b2)
